# baseline (speedup 1.0000x reference)
import functools

import jax
import jax.numpy as jnp
from jax import lax
from jax.experimental import pallas as pl
from jax.experimental.pallas import tpu as pltpu
from jax.experimental.pallas import tpu_sc as plsc

_BATCH = 1024
_SEQ = 200
_HIDDEN = 128
_NVEC = _HIDDEN // 16

_info = plsc.get_sparse_core_info()
_NC = _info.num_cores
_NS = _info.num_subcores
_NW = _NC * _NS
_B_PER_W = _BATCH // _NW


def _row_add(xbuf, obuf, cond_v, l):
    cvecs = [cond_v[l, pl.ds(c * 16, 16)] for c in range(_NVEC)]

    def s_body(s, _):
        for u in range(2):
            for c in range(_NVEC):
                sl = pl.ds(c * 16, 16)
                obuf[2 * s + u, sl] = xbuf[2 * s + u, sl] + cvecs[c]
        return 0

    lax.fori_loop(0, _SEQ // 2, s_body, 0, unroll=False)


def _sc_kernel_body(idx_hbm, x_hbm, table_hbm, out_hbm,
                    idx_v, cond_v, xb0, xb1, ob0, ob1,
                    sem_g, si0, si1, so0, so1):
    wid = lax.axis_index("s") * _NC + lax.axis_index("c")
    base = wid * _B_PER_W

    pltpu.sync_copy(idx_hbm.at[pl.ds(base, _B_PER_W)], idx_v)
    pltpu.async_copy(table_hbm.at[idx_v], cond_v, sem_g).wait()

    pltpu.async_copy(x_hbm.at[base], xb0, si0)

    def g_body(g, _):
        l0 = 2 * g
        l1 = 2 * g + 1
        pltpu.async_copy(x_hbm.at[base + l1], xb1, si1)
        pltpu.make_async_copy(x_hbm.at[base], xb0, si0).wait()

        @pl.when(g > 0)
        def _():
            pltpu.make_async_copy(ob0, out_hbm.at[base], so0).wait()

        _row_add(xb0, ob0, cond_v, l0)
        pltpu.async_copy(ob0, out_hbm.at[base + l0], so0)

        @pl.when(g < _B_PER_W // 2 - 1)
        def _():
            pltpu.async_copy(x_hbm.at[base + l1 + 1], xb0, si0)

        pltpu.make_async_copy(x_hbm.at[base], xb1, si1).wait()

        @pl.when(g > 0)
        def _():
            pltpu.make_async_copy(ob1, out_hbm.at[base], so1).wait()

        _row_add(xb1, ob1, cond_v, l1)
        pltpu.async_copy(ob1, out_hbm.at[base + l1], so1)
        return 0

    lax.fori_loop(0, _B_PER_W // 2, g_body, 0, unroll=False)

    pltpu.make_async_copy(ob0, out_hbm.at[base], so0).wait()
    pltpu.make_async_copy(ob1, out_hbm.at[base], so1).wait()


def _sc_perturb_add(pert_ids, x, emb):
    mesh = plsc.VectorSubcoreMesh(core_axis_name="c", subcore_axis_name="s")
    return functools.partial(
        pl.kernel,
        mesh=mesh,
        out_type=jax.ShapeDtypeStruct((_BATCH, _SEQ, _HIDDEN), jnp.float32),
        scratch_types=[
            pltpu.VMEM((_B_PER_W,), jnp.int32),
            pltpu.VMEM((_B_PER_W, _HIDDEN), jnp.float32),
            pltpu.VMEM((_SEQ, _HIDDEN), jnp.float32),
            pltpu.VMEM((_SEQ, _HIDDEN), jnp.float32),
            pltpu.VMEM((_SEQ, _HIDDEN), jnp.float32),
            pltpu.VMEM((_SEQ, _HIDDEN), jnp.float32),
            pltpu.SemaphoreType.DMA,
            pltpu.SemaphoreType.DMA,
            pltpu.SemaphoreType.DMA,
            pltpu.SemaphoreType.DMA,
            pltpu.SemaphoreType.DMA,
        ],
    )(_sc_kernel_body)(pert_ids, x, emb)


def kernel(x, pert_ids, emb):
    return _sc_perturb_add(pert_ids.astype(jnp.int32), x, emb)
